# Initial kernel scaffold; baseline (speedup 1.0000x reference)
import jax
import jax.numpy as jnp
from jax.experimental import pallas as pl


def kernel(input_ids, table):
    raise NotImplementedError("write your pallas kernel here")


# speedup vs baseline: 1.6437x; 1.6437x over previous
import functools

import jax
import jax.numpy as jnp
from jax import lax
from jax.experimental import pallas as pl
from jax.experimental.pallas import tpu as pltpu
from jax.experimental.pallas import tpu_sc as plsc

D = 128
ROWS = 4096
COLS = 200
B = ROWS * COLS

_info = plsc.get_sparse_core_info()
NC = _info.num_cores
NS = _info.num_subcores
NW = NC * NS
PER_W = B // NW

RPG = 128
K = 4
CHUNK = K * RPG
NCHUNK = PER_W // CHUNK
GROUPS = PER_W // RPG

_mesh = plsc.VectorSubcoreMesh(core_axis_name="c", subcore_axis_name="s")


@functools.partial(
    pl.kernel,
    mesh=_mesh,
    out_type=jax.ShapeDtypeStruct((B, D), jnp.float32),
    scratch_types=[
        pltpu.VMEM((K, RPG), jnp.int32),
        pltpu.VMEM((CHUNK, D), jnp.float32),
        pltpu.SemaphoreType.DMA,
    ],
)
def _emb_lookup(ids_hbm, table_hbm, out_hbm, idx_v, rows_v, sem):
    wid = lax.axis_index("s") * NC + lax.axis_index("c")
    base = wid * PER_W

    def body(i, carry):
        pltpu.sync_copy(ids_hbm.at[wid, pl.ds(i * K, K)], idx_v)
        cps = [
            pltpu.async_copy(
                table_hbm.at[idx_v.at[j]],
                rows_v.at[pl.ds(j * RPG, RPG)],
                sem,
            )
            for j in range(K)
        ]
        for cp in cps:
            cp.wait()
        pltpu.sync_copy(rows_v, out_hbm.at[pl.ds(base + i * CHUNK, CHUNK)])
        return carry

    lax.fori_loop(0, NCHUNK, body, 0)


def kernel(input_ids, table):
    ids = input_ids.reshape(NW, GROUPS, RPG).astype(jnp.int32)
    out = _emb_lookup(ids, table)
    return out.reshape(ROWS, COLS, D)


# speedup vs baseline: 1.8472x; 1.1238x over previous
import functools

import jax
import jax.numpy as jnp
from jax import lax
from jax.experimental import pallas as pl
from jax.experimental.pallas import tpu as pltpu
from jax.experimental.pallas import tpu_sc as plsc

D = 128
ROWS = 4096
COLS = 200
B = ROWS * COLS

_info = plsc.get_sparse_core_info()
NC = _info.num_cores
NS = _info.num_subcores
NW = NC * NS
PER_W = B // NW

RPG = 128
K = 2
CHUNK = K * RPG
NCHUNK = PER_W // CHUNK
GROUPS = PER_W // RPG

_mesh = plsc.VectorSubcoreMesh(core_axis_name="c", subcore_axis_name="s")


@functools.partial(
    pl.kernel,
    mesh=_mesh,
    out_type=jax.ShapeDtypeStruct((B, D), jnp.float32),
    scratch_types=[
        pltpu.VMEM((GROUPS, RPG), jnp.int32),
        pltpu.VMEM((2, CHUNK, D), jnp.float32),
        pltpu.SemaphoreType.DMA,
        pltpu.SemaphoreType.DMA,
        pltpu.SemaphoreType.DMA,
        pltpu.SemaphoreType.DMA,
    ],
)
def _emb_lookup(ids_hbm, table_hbm, out_hbm, idx_v, rows_v, g0, g1, o0, o1):
    wid = lax.axis_index("s") * NC + lax.axis_index("c")
    base = wid * PER_W
    gsem = (g0, g1)
    osem = (o0, o1)

    pltpu.sync_copy(ids_hbm.at[wid], idx_v)

    def fire_gather(i, s):
        for j in range(K):
            pltpu.async_copy(
                table_hbm.at[idx_v.at[i * K + j]],
                rows_v.at[s, pl.ds(j * RPG, RPG)],
                gsem[s],
            )

    def drain_gather(s):
        pltpu.make_async_copy(
            table_hbm.at[pl.ds(0, CHUNK)], rows_v.at[s], gsem[s]
        ).wait()

    def fire_out(i, s):
        pltpu.async_copy(
            rows_v.at[s], out_hbm.at[pl.ds(base + i * CHUNK, CHUNK)], osem[s]
        )

    def drain_out(s):
        pltpu.make_async_copy(
            rows_v.at[s], out_hbm.at[pl.ds(0, CHUNK)], osem[s]
        ).wait()

    fire_gather(0, 0)
    fire_gather(1, 1)
    drain_gather(0)
    fire_out(0, 0)

    def body(u, carry):
        i1 = 2 * u + 1
        drain_out(0)
        fire_gather(i1 + 1, 0)
        drain_gather(1)
        fire_out(i1, 1)
        i2 = i1 + 1
        drain_out(1)
        fire_gather(i2 + 1, 1)
        drain_gather(0)
        fire_out(i2, 0)
        return carry

    lax.fori_loop(0, (NCHUNK - 2) // 2, body, 0)

    drain_out(0)
    drain_gather(1)
    fire_out(NCHUNK - 1, 1)
    drain_out(1)


def kernel(input_ids, table):
    ids = input_ids.reshape(NW, GROUPS, RPG).astype(jnp.int32)
    out = _emb_lookup(ids, table)
    return out.reshape(ROWS, COLS, D)


# speedup vs baseline: 1.8544x; 1.0039x over previous
import functools

import jax
import jax.numpy as jnp
from jax import lax
from jax.experimental import pallas as pl
from jax.experimental.pallas import tpu as pltpu
from jax.experimental.pallas import tpu_sc as plsc

D = 128
ROWS = 4096
COLS = 200
B = ROWS * COLS

_info = plsc.get_sparse_core_info()
NC = _info.num_cores
NS = _info.num_subcores
NW = NC * NS
PER_W = B // NW

RPG = 128
K = 2
CHUNK = K * RPG
NCHUNK = PER_W // CHUNK
GROUPS = PER_W // RPG

_mesh = plsc.VectorSubcoreMesh(core_axis_name="c", subcore_axis_name="s")


@functools.partial(
    pl.kernel,
    mesh=_mesh,
    out_type=jax.ShapeDtypeStruct((B, D), jnp.float32),
    scratch_types=[
        pltpu.VMEM((GROUPS, RPG), jnp.int32),
        pltpu.VMEM((3, CHUNK, D), jnp.float32),
        pltpu.SemaphoreType.DMA,
        pltpu.SemaphoreType.DMA,
        pltpu.SemaphoreType.DMA,
        pltpu.SemaphoreType.DMA,
        pltpu.SemaphoreType.DMA,
        pltpu.SemaphoreType.DMA,
    ],
)
def _emb_lookup(ids_hbm, table_hbm, out_hbm, idx_v, rows_v, g0, g1, g2, o0, o1, o2):
    wid = lax.axis_index("s") * NC + lax.axis_index("c")
    base = wid * PER_W
    gsem = (g0, g1, g2)
    osem = (o0, o1, o2)

    pltpu.sync_copy(ids_hbm.at[wid], idx_v)

    def fire_gather(i, s):
        for j in range(K):
            pltpu.async_copy(
                table_hbm.at[idx_v.at[i * K + j]],
                rows_v.at[s, pl.ds(j * RPG, RPG)],
                gsem[s],
            )

    def drain_gather(s):
        pltpu.make_async_copy(
            table_hbm.at[pl.ds(0, CHUNK)], rows_v.at[s], gsem[s]
        ).wait()

    def fire_out(i, s):
        pltpu.async_copy(
            rows_v.at[s], out_hbm.at[pl.ds(base + i * CHUNK, CHUNK)], osem[s]
        )

    def drain_out(s):
        pltpu.make_async_copy(
            rows_v.at[s], out_hbm.at[pl.ds(0, CHUNK)], osem[s]
        ).wait()

    def step(i, s):
        drain_out((s + 2) % 3)
        fire_gather(i + 2, (s + 2) % 3)
        drain_gather(s)
        fire_out(i, s)

    fire_gather(0, 0)
    fire_gather(1, 1)
    fire_gather(2, 2)
    drain_gather(0)
    fire_out(0, 0)
    step(1, 1)

    def body(u, carry):
        i = 3 * u + 2
        step(i, 2)
        step(i + 1, 0)
        step(i + 2, 1)
        return carry

    lax.fori_loop(0, (NCHUNK - 4) // 3, body, 0)

    drain_out(1)
    drain_gather(2)
    fire_out(NCHUNK - 2, 2)
    drain_out(2)
    drain_gather(0)
    fire_out(NCHUNK - 1, 0)
    drain_out(0)


def kernel(input_ids, table):
    ids = input_ids.reshape(NW, GROUPS, RPG).astype(jnp.int32)
    out = _emb_lookup(ids, table)
    return out.reshape(ROWS, COLS, D)
